# baseline (speedup 1.0000x reference)
import jax
import jax.numpy as jnp
from jax import lax
from jax.experimental import pallas as pl
from jax.experimental.pallas import tpu as pltpu
from jax.experimental.pallas import tpu_sc as plsc

N = 10000
E = 320000
G = 128
EMB = 32
FIN = 128
RW = 48

NC = 2
NS = 16
NW = NC * NS
CHUNK = 128
NBUF = 8
KCH = 80
NGRP = KCH // NBUF
SLAB = NW * KCH * CHUNK
NPAD = 10240
ROWS_PER_SUB = NPAD // NS
OFFP = 152
SEGS_PER_W = G // NW

_mesh = plsc.VectorSubcoreMesh(
    core_axis_name="c", subcore_axis_name="s", num_cores=NC, num_subcores=NS)


def _agg_body(ht, src, dst, zeros, out, src_v, dst_v, acc, sht, *bufsem):
    rows = bufsem[:NBUF]
    sems = bufsem[NBUF:2 * NBUF]
    ssems = bufsem[2 * NBUF:]
    c = lax.axis_index("c")
    s = lax.axis_index("s")
    wid = c * NS + s
    pltpu.sync_copy(zeros.at[pl.ds(s * ROWS_PER_SUB, ROWS_PER_SUB)],
                    acc.at[pl.ds(s * ROWS_PER_SUB, ROWS_PER_SUB)])
    pltpu.sync_copy(ht.at[pl.ds(s * ROWS_PER_SUB, ROWS_PER_SUB)],
                    sht.at[pl.ds(s * ROWS_PER_SUB, ROWS_PER_SUB)])
    pltpu.sync_copy(src.at[wid], src_v)
    pltpu.sync_copy(dst.at[wid], dst_v)
    plsc.subcore_barrier()

    def body(grp, carry):
        k0 = grp * NBUF
        descs = []
        for b in range(NBUF):
            descs.append(
                pltpu.async_copy(sht.at[src_v.at[k0 + b]], rows[b], sems[b]))
        for b in range(NBUF):
            descs[b].wait()
            pltpu.sync_copy(rows[b], acc.at[dst_v.at[k0 + b]], add=True)
        return carry

    lax.fori_loop(0, NGRP, body, 0)
    plsc.subcore_barrier()
    pltpu.sync_copy(acc.at[pl.ds(s * ROWS_PER_SUB, ROWS_PER_SUB)],
                    out.at[c, pl.ds(s * ROWS_PER_SUB, ROWS_PER_SUB)])


def _make_agg(interpret=False):
    return pl.kernel(
        _agg_body,
        out_type=jax.ShapeDtypeStruct((NC, NPAD, RW), jnp.float32),
        mesh=_mesh,
        scratch_types=[
            pltpu.VMEM((KCH, CHUNK), jnp.int32),
            pltpu.VMEM((KCH, CHUNK), jnp.int32),
            pltpu.VMEM_SHARED((NPAD, RW), jnp.float32),
            pltpu.VMEM_SHARED((NPAD, RW), jnp.float32),
        ] + [pltpu.VMEM((CHUNK, RW), jnp.float32)] * NBUF
          + [pltpu.SemaphoreType.DMA] * (2 * NBUF),
        compiler_params=pltpu.CompilerParams(use_tc_tiling_on_sc=False),
        interpret=interpret,
    )


def _pool_body(h1, h2, h3, h4, h5, h6, h7, off_hbm, out,
               b1, b2, b3, b4, b5, b6, b7, pooled_v, off_v):
    bufs = (b1, b2, b3, b4, b5, b6, b7)
    hs = (h1, h2, h3, h4, h5, h6, h7)
    c = lax.axis_index("c")
    s = lax.axis_index("s")
    wid = c * NS + s
    pltpu.sync_copy(off_hbm, off_v)

    neg_inf = jnp.full((16,), -jnp.inf, jnp.float32)
    zero = jnp.zeros((16,), jnp.float32)

    for j in range(SEGS_PER_W):
        g = wid * SEGS_PER_W + j
        offs = off_v[pl.ds(g, 16)]
        start = offs[0]
        end = offs[1]
        cnt = end - start
        nch = (cnt + (CHUNK - 1)) >> 7

        def chunk_body(ci, carry):
            base = start + ci * CHUNK
            rem = jnp.minimum(CHUNK, cnt - ci * CHUNK)
            for ll in range(7):
                pltpu.sync_copy(hs[ll].at[pl.ds(base, CHUNK)], bufs[ll])

            def row_body(r, carry2):
                mx, sm = carry2
                mx2, sm2 = [], []
                for ll in range(7):
                    for hh in range(2):
                        v = bufs[ll][r, pl.ds(16 * hh, 16)]
                        mx2.append(jnp.maximum(mx[2 * ll + hh], v))
                        sm2.append(sm[2 * ll + hh] + v)
                return tuple(mx2), tuple(sm2)

            return lax.fori_loop(0, rem, row_body, carry)

        init = (tuple(neg_inf for _ in range(14)),
                tuple(zero for _ in range(14)))
        mx, sm = lax.fori_loop(0, nch, chunk_body, init)

        cntv = jnp.full((16,), 1.0, jnp.float32) * cnt.astype(jnp.float32)
        inv = 1.0 / jnp.maximum(cntv, 1.0)
        for ll in range(7):
            for hh in range(2):
                pooled_v[j, pl.ds(64 * ll + 16 * hh, 16)] = mx[2 * ll + hh]
                pooled_v[j, pl.ds(64 * ll + 32 + 16 * hh, 16)] = (
                    sm[2 * ll + hh] * inv)

    pltpu.sync_copy(pooled_v, out.at[pl.ds(wid * SEGS_PER_W, SEGS_PER_W)])


def _make_pool(interpret=False):
    return pl.kernel(
        _pool_body,
        out_type=jax.ShapeDtypeStruct((G, 448), jnp.float32),
        mesh=_mesh,
        scratch_types=[pltpu.VMEM((CHUNK, EMB), jnp.float32)] * 7 + [
            pltpu.VMEM((SEGS_PER_W, 448), jnp.float32),
            pltpu.VMEM((OFFP,), jnp.int32),
        ],
        compiler_params=pltpu.CompilerParams(use_tc_tiling_on_sc=False),
        interpret=interpret,
    )


def _prep_tc(deg2_ref, xp_ref, w0_ref, batch_ref,
             dis_ref, dis2_ref, g1_ref, ht1_ref, off_ref):
    deg = (deg2_ref[0] + deg2_ref[1])[:, 0:1] + 1.0
    valid = lax.broadcasted_iota(jnp.int32, (NPAD, 1), 0) < N
    dis = jnp.where(valid, lax.rsqrt(deg), 0.0)
    dis_ref[...] = dis
    dis2_ref[...] = dis * dis
    g1 = jnp.dot(xp_ref[...], w0_ref[...], preferred_element_type=jnp.float32)
    g1_ref[...] = g1
    ht1_ref[...] = dis * g1
    gid = lax.broadcasted_iota(jnp.int32, (OFFP, 1), 0)
    cmp = (batch_ref[...] < gid).astype(jnp.float32)
    off_ref[...] = jnp.sum(cmp, axis=1, keepdims=True).astype(jnp.int32)


def _combine_tc(acc2_ref, g_ref, dis_ref, dis2_ref, b_ref, wn_ref,
                h_ref, gn_ref, htn_ref):
    h = (dis_ref[...] * (acc2_ref[0] + acc2_ref[1])
         + dis2_ref[...] * g_ref[...] + b_ref[...])
    h_ref[...] = h
    gn = jnp.dot(h, wn_ref[...], preferred_element_type=jnp.float32)
    gn_ref[...] = gn
    htn_ref[...] = dis_ref[...] * gn


def _combine7_tc(acc2_ref, g_ref, dis_ref, dis2_ref, b_ref, h_ref):
    h_ref[...] = (dis_ref[...] * (acc2_ref[0] + acc2_ref[1])
                  + dis2_ref[...] * g_ref[...] + b_ref[...])


def _mlp_tc(pooled_ref, wout_ref, bout_ref, wout2_ref, bout2_ref,
            sig_ref, log_ref):
    hid = jnp.dot(pooled_ref[...], wout_ref[...],
                  preferred_element_type=jnp.float32) + bout_ref[...]
    hid = jnp.maximum(hid, 0.0)
    logits = jnp.dot(hid, wout2_ref[...],
                     preferred_element_type=jnp.float32) + bout2_ref[...]
    log_ref[...] = logits
    sig_ref[...] = jax.nn.sigmoid(logits)


def _tc_call(body, out_shapes, interpret=False):
    return pl.pallas_call(body, out_shape=out_shapes, interpret=interpret)


def _run(x, edge_index, batch_index, Ws, bs, Wout, bout, Wout2, bout2,
         interpret=False):
    f32 = jnp.float32
    agg = _make_agg(interpret)
    pool = _make_pool(interpret)

    padlen = SLAB - E
    src = jnp.concatenate(
        [edge_index[0],
         jnp.full((padlen,), N, jnp.int32)]).reshape(NW, KCH, CHUNK)
    dst = jnp.concatenate(
        [edge_index[1],
         jnp.full((padlen,), N, jnp.int32)]).reshape(NW, KCH, CHUNK)

    zeros = jnp.zeros((NPAD, RW), f32)
    row_valid = (jnp.arange(NPAD, dtype=jnp.int32) < N)[:, None]
    valid_ones = jnp.where(row_valid, 1.0, 0.0) * jnp.ones((NPAD, RW), f32)
    xp = jnp.pad(x, ((0, NPAD - N), (0, 0)))
    batch2d = batch_index.reshape(1, N)

    deg2 = agg(valid_ones, src, dst, zeros)

    dis, dis2, g1, ht1, off2d = _tc_call(
        _prep_tc,
        (jax.ShapeDtypeStruct((NPAD, 1), f32),
         jax.ShapeDtypeStruct((NPAD, 1), f32),
         jax.ShapeDtypeStruct((NPAD, EMB), f32),
         jax.ShapeDtypeStruct((NPAD, EMB), f32),
         jax.ShapeDtypeStruct((OFFP, 1), jnp.int32)),
        interpret)(deg2, xp, Ws[0], batch2d)
    off = off2d.reshape(OFFP)

    hs = []
    g, ht = g1, ht1
    for k in range(7):
        acc2 = agg(jnp.pad(ht, ((0, 0), (0, RW - EMB))), src, dst, zeros)[:, :, :EMB]
        bk = bs[k].reshape(1, EMB)
        if k < 6:
            h, g, ht = _tc_call(
                _combine_tc,
                (jax.ShapeDtypeStruct((NPAD, EMB), f32),
                 jax.ShapeDtypeStruct((NPAD, EMB), f32),
                 jax.ShapeDtypeStruct((NPAD, EMB), f32)),
                interpret)(acc2, g, dis, dis2, bk, Ws[k + 1])
        else:
            h = _tc_call(
                _combine7_tc,
                jax.ShapeDtypeStruct((NPAD, EMB), f32),
                interpret)(acc2, g, dis, dis2, bk)
        hs.append(h)

    pooled = pool(hs[0], hs[1], hs[2], hs[3], hs[4], hs[5], hs[6], off)

    sig, logits = _tc_call(
        _mlp_tc,
        (jax.ShapeDtypeStruct((G, 1), f32),
         jax.ShapeDtypeStruct((G, 1), f32)),
        interpret)(pooled, Wout, bout.reshape(1, 448), Wout2,
                   bout2.reshape(1, 1))
    return sig, logits


def kernel(x, edge_index, batch_index, W0, b0, W1, b1, W2, b2, W3, b3,
           W4, b4, W5, b5, W6, b6, Wout, bout, Wout2, bout2):
    return _run(x, edge_index, batch_index,
                (W0, W1, W2, W3, W4, W5, W6),
                (b0, b1, b2, b3, b4, b5, b6),
                Wout, bout, Wout2, bout2)


# speedup vs baseline: 1.3117x; 1.3117x over previous
import jax
import jax.numpy as jnp
from jax import lax
from jax.experimental import pallas as pl
from jax.experimental.pallas import tpu as pltpu
from jax.experimental.pallas import tpu_sc as plsc

N = 10000
E = 320000
G = 128
EMB = 32
FIN = 128
RW = 32

NC = 2
NS = 16
NW = NC * NS
CHUNK = 128
NBUF = 8
KCH = 80
NGRP = KCH // NBUF
SLAB = NW * KCH * CHUNK
NPAD = 10240
ROWS_PER_SUB = NPAD // NS
OFFP = 152
SEGS_PER_W = G // NW

_mesh = plsc.VectorSubcoreMesh(
    core_axis_name="c", subcore_axis_name="s", num_cores=NC, num_subcores=NS)


def _agg_body(ht, src, dst, zeros, out, src_v, dst_v, acc, sht, *bufsem):
    rows = bufsem[:NBUF]
    sems = bufsem[NBUF:2 * NBUF]
    ssems = bufsem[2 * NBUF:]
    c = lax.axis_index("c")
    s = lax.axis_index("s")
    wid = c * NS + s
    pltpu.sync_copy(zeros.at[pl.ds(s * ROWS_PER_SUB, ROWS_PER_SUB)],
                    acc.at[pl.ds(s * ROWS_PER_SUB, ROWS_PER_SUB)])
    pltpu.sync_copy(ht.at[pl.ds(s * ROWS_PER_SUB, ROWS_PER_SUB)],
                    sht.at[pl.ds(s * ROWS_PER_SUB, ROWS_PER_SUB)])
    pltpu.sync_copy(src.at[wid], src_v)
    pltpu.sync_copy(dst.at[wid], dst_v)
    plsc.subcore_barrier()

    def body(grp, carry):
        k0 = grp * NBUF
        descs = []
        for b in range(NBUF):
            descs.append(
                pltpu.async_copy(sht.at[src_v.at[k0 + b]], rows[b], sems[b]))
        for b in range(NBUF):
            descs[b].wait()
            pltpu.sync_copy(rows[b], acc.at[dst_v.at[k0 + b]], add=True)
        return carry

    lax.fori_loop(0, NGRP, body, 0)
    plsc.subcore_barrier()
    pltpu.sync_copy(acc.at[pl.ds(s * ROWS_PER_SUB, ROWS_PER_SUB)],
                    out.at[c, pl.ds(s * ROWS_PER_SUB, ROWS_PER_SUB)])


def _make_agg(interpret=False):
    return pl.kernel(
        _agg_body,
        out_type=jax.ShapeDtypeStruct((NC, NPAD, RW), jnp.float32),
        mesh=_mesh,
        scratch_types=[
            pltpu.VMEM((KCH, CHUNK), jnp.int32),
            pltpu.VMEM((KCH, CHUNK), jnp.int32),
            pltpu.VMEM_SHARED((NPAD, RW), jnp.float32),
            pltpu.VMEM_SHARED((NPAD, RW), jnp.float32),
        ] + [pltpu.VMEM((CHUNK, RW), jnp.float32)] * NBUF
          + [pltpu.SemaphoreType.DMA] * (2 * NBUF),
        compiler_params=pltpu.CompilerParams(use_tc_tiling_on_sc=False),
        interpret=interpret,
    )


def _deg_body(dst, zeros1, out, dst_v, ones_v, dacc):
    c = lax.axis_index("c")
    s = lax.axis_index("s")
    wid = c * NS + s
    pltpu.sync_copy(dst.at[wid], dst_v)
    pltpu.sync_copy(zeros1.at[pl.ds(s * ROWS_PER_SUB, ROWS_PER_SUB)],
                    dacc.at[pl.ds(s * ROWS_PER_SUB, ROWS_PER_SUB)])
    one16 = jnp.full((16,), 1.0, jnp.float32)
    for r in range(CHUNK // 16):
        ones_v[pl.ds(r * 16, 16)] = one16
    plsc.subcore_barrier()

    def body(k, carry):
        pltpu.sync_copy(ones_v, dacc.at[dst_v.at[k]], add=True)
        return carry

    lax.fori_loop(0, KCH, body, 0)
    plsc.subcore_barrier()
    pltpu.sync_copy(dacc.at[pl.ds(s * ROWS_PER_SUB, ROWS_PER_SUB)],
                    out.at[c, pl.ds(s * ROWS_PER_SUB, ROWS_PER_SUB)])


def _make_deg(interpret=False):
    return pl.kernel(
        _deg_body,
        out_type=jax.ShapeDtypeStruct((NC, NPAD), jnp.float32),
        mesh=_mesh,
        scratch_types=[
            pltpu.VMEM((KCH, CHUNK), jnp.int32),
            pltpu.VMEM((CHUNK,), jnp.float32),
            pltpu.VMEM_SHARED((NPAD,), jnp.float32),
        ],
        compiler_params=pltpu.CompilerParams(use_tc_tiling_on_sc=False),
        interpret=interpret,
    )


def _pool_body(h1, h2, h3, h4, h5, h6, h7, off_hbm, out,
               b1, b2, b3, b4, b5, b6, b7, pooled_v, off_v, *psems):
    bufs = (b1, b2, b3, b4, b5, b6, b7)
    hs = (h1, h2, h3, h4, h5, h6, h7)
    c = lax.axis_index("c")
    s = lax.axis_index("s")
    wid = c * NS + s
    pltpu.sync_copy(off_hbm, off_v)

    neg_inf = jnp.full((16,), -jnp.inf, jnp.float32)
    zero = jnp.zeros((16,), jnp.float32)

    for j in range(SEGS_PER_W):
        g = wid * SEGS_PER_W + j
        offs = off_v[pl.ds(g, 16)]
        start = offs[0]
        end = offs[1]
        cnt = end - start
        nch = (cnt + (CHUNK - 1)) >> 7

        def chunk_body(ci, carry):
            base = start + ci * CHUNK
            rem = jnp.minimum(CHUNK, cnt - ci * CHUNK)
            descs = [
                pltpu.async_copy(hs[ll].at[pl.ds(base, CHUNK)], bufs[ll],
                                 psems[ll])
                for ll in range(7)]
            for d in descs:
                d.wait()

            def row_body(r, carry2):
                mx, sm = carry2
                mx2, sm2 = [], []
                for ll in range(7):
                    for hh in range(2):
                        v = bufs[ll][r, pl.ds(16 * hh, 16)]
                        mx2.append(jnp.maximum(mx[2 * ll + hh], v))
                        sm2.append(sm[2 * ll + hh] + v)
                return tuple(mx2), tuple(sm2)

            return lax.fori_loop(0, rem, row_body, carry)

        init = (tuple(neg_inf for _ in range(14)),
                tuple(zero for _ in range(14)))
        mx, sm = lax.fori_loop(0, nch, chunk_body, init)

        cntv = jnp.full((16,), 1.0, jnp.float32) * cnt.astype(jnp.float32)
        inv = 1.0 / jnp.maximum(cntv, 1.0)
        for ll in range(7):
            for hh in range(2):
                pooled_v[j, pl.ds(64 * ll + 16 * hh, 16)] = mx[2 * ll + hh]
                pooled_v[j, pl.ds(64 * ll + 32 + 16 * hh, 16)] = (
                    sm[2 * ll + hh] * inv)

    pltpu.sync_copy(pooled_v, out.at[pl.ds(wid * SEGS_PER_W, SEGS_PER_W)])


def _make_pool(interpret=False):
    return pl.kernel(
        _pool_body,
        out_type=jax.ShapeDtypeStruct((G, 448), jnp.float32),
        mesh=_mesh,
        scratch_types=[pltpu.VMEM((CHUNK, EMB), jnp.float32)] * 7 + [
            pltpu.VMEM((SEGS_PER_W, 448), jnp.float32),
            pltpu.VMEM((OFFP,), jnp.int32),
        ] + [pltpu.SemaphoreType.DMA] * 7,
        compiler_params=pltpu.CompilerParams(use_tc_tiling_on_sc=False),
        interpret=interpret,
    )


def _prep_tc(deg2_ref, xp_ref, w0_ref, batch_ref,
             dis_ref, dis2_ref, g1_ref, ht1_ref, off_ref):
    deg = (deg2_ref[0] + deg2_ref[1]) + 1.0
    valid = lax.broadcasted_iota(jnp.int32, (NPAD, 1), 0) < N
    dis = jnp.where(valid, lax.rsqrt(deg), 0.0)
    dis_ref[...] = dis
    dis2_ref[...] = dis * dis
    g1 = jnp.dot(xp_ref[...], w0_ref[...], preferred_element_type=jnp.float32)
    g1_ref[...] = g1
    ht1_ref[...] = dis * g1
    gid = lax.broadcasted_iota(jnp.int32, (OFFP, 1), 0)
    cmp = (batch_ref[...] < gid).astype(jnp.float32)
    off_ref[...] = jnp.sum(cmp, axis=1, keepdims=True).astype(jnp.int32)


def _combine_tc(acc2_ref, g_ref, dis_ref, dis2_ref, b_ref, wn_ref,
                h_ref, gn_ref, htn_ref):
    h = (dis_ref[...] * (acc2_ref[0] + acc2_ref[1])
         + dis2_ref[...] * g_ref[...] + b_ref[...])
    h_ref[...] = h
    gn = jnp.dot(h, wn_ref[...], preferred_element_type=jnp.float32)
    gn_ref[...] = gn
    htn_ref[...] = dis_ref[...] * gn


def _combine7_tc(acc2_ref, g_ref, dis_ref, dis2_ref, b_ref, h_ref):
    h_ref[...] = (dis_ref[...] * (acc2_ref[0] + acc2_ref[1])
                  + dis2_ref[...] * g_ref[...] + b_ref[...])


def _mlp_tc(pooled_ref, wout_ref, bout_ref, wout2_ref, bout2_ref,
            sig_ref, log_ref):
    hid = jnp.dot(pooled_ref[...], wout_ref[...],
                  preferred_element_type=jnp.float32) + bout_ref[...]
    hid = jnp.maximum(hid, 0.0)
    logits = jnp.dot(hid, wout2_ref[...],
                     preferred_element_type=jnp.float32) + bout2_ref[...]
    log_ref[...] = logits
    sig_ref[...] = jax.nn.sigmoid(logits)


def _tc_call(body, out_shapes, interpret=False):
    return pl.pallas_call(body, out_shape=out_shapes, interpret=interpret)


def _run(x, edge_index, batch_index, Ws, bs, Wout, bout, Wout2, bout2,
         interpret=False):
    f32 = jnp.float32
    agg = _make_agg(interpret)
    pool = _make_pool(interpret)

    padlen = SLAB - E
    src = jnp.concatenate(
        [edge_index[0],
         jnp.full((padlen,), N, jnp.int32)]).reshape(NW, KCH, CHUNK)
    dst = jnp.concatenate(
        [edge_index[1],
         jnp.full((padlen,), N, jnp.int32)]).reshape(NW, KCH, CHUNK)

    zeros = jnp.zeros((NPAD, RW), f32)
    row_valid = (jnp.arange(NPAD, dtype=jnp.int32) < N)[:, None]
    valid_ones = jnp.where(row_valid, 1.0, 0.0) * jnp.ones((NPAD, RW), f32)
    xp = jnp.pad(x, ((0, NPAD - N), (0, 0)))
    batch2d = batch_index.reshape(1, N)

    deg2 = _make_deg(interpret)(dst, jnp.zeros((NPAD,), f32)).reshape(
        NC, NPAD, 1)

    dis, dis2, g1, ht1, off2d = _tc_call(
        _prep_tc,
        (jax.ShapeDtypeStruct((NPAD, 1), f32),
         jax.ShapeDtypeStruct((NPAD, 1), f32),
         jax.ShapeDtypeStruct((NPAD, EMB), f32),
         jax.ShapeDtypeStruct((NPAD, EMB), f32),
         jax.ShapeDtypeStruct((OFFP, 1), jnp.int32)),
        interpret)(deg2, xp, Ws[0], batch2d)
    off = off2d.reshape(OFFP)

    hs = []
    g, ht = g1, ht1
    for k in range(7):
        acc2 = agg(ht, src, dst, zeros)[:, :, :EMB]
        bk = bs[k].reshape(1, EMB)
        if k < 6:
            h, g, ht = _tc_call(
                _combine_tc,
                (jax.ShapeDtypeStruct((NPAD, EMB), f32),
                 jax.ShapeDtypeStruct((NPAD, EMB), f32),
                 jax.ShapeDtypeStruct((NPAD, EMB), f32)),
                interpret)(acc2, g, dis, dis2, bk, Ws[k + 1])
        else:
            h = _tc_call(
                _combine7_tc,
                jax.ShapeDtypeStruct((NPAD, EMB), f32),
                interpret)(acc2, g, dis, dis2, bk)
        hs.append(h)

    pooled = pool(hs[0], hs[1], hs[2], hs[3], hs[4], hs[5], hs[6], off)

    sig, logits = _tc_call(
        _mlp_tc,
        (jax.ShapeDtypeStruct((G, 1), f32),
         jax.ShapeDtypeStruct((G, 1), f32)),
        interpret)(pooled, Wout, bout.reshape(1, 448), Wout2,
                   bout2.reshape(1, 1))
    return sig, logits


def kernel(x, edge_index, batch_index, W0, b0, W1, b1, W2, b2, W3, b3,
           W4, b4, W5, b5, W6, b6, Wout, bout, Wout2, bout2):
    return _run(x, edge_index, batch_index,
                (W0, W1, W2, W3, W4, W5, W6),
                (b0, b1, b2, b3, b4, b5, b6),
                Wout, bout, Wout2, bout2)
